# Initial kernel scaffold; baseline (speedup 1.0000x reference)
import jax
import jax.numpy as jnp
from jax.experimental import pallas as pl


def kernel(input_ids, embedding_table):
    raise NotImplementedError("write your pallas kernel here")


# speedup vs baseline: 1.4414x; 1.4414x over previous
import functools

import jax
import jax.numpy as jnp
from jax import lax
from jax.experimental import pallas as pl
from jax.experimental.pallas import tpu as pltpu
from jax.experimental.pallas import tpu_sc as plsc

D_MODEL = 768
N_TOKENS = 4 * 2048
NUM_WORKERS = 32
ROWS_PER_WORKER = N_TOKENS // NUM_WORKERS
CHUNK = 64
NCHUNK = ROWS_PER_WORKER // CHUNK
NBUF = 2

_MESH = plsc.VectorSubcoreMesh(core_axis_name="c", subcore_axis_name="s")


@functools.partial(
    pl.kernel,
    mesh=_MESH,
    out_type=jax.ShapeDtypeStruct((N_TOKENS, D_MODEL), jnp.float32),
    scratch_types=[
        pltpu.VMEM((NCHUNK, CHUNK), jnp.int32),
        pltpu.VMEM((NBUF, CHUNK, D_MODEL), jnp.float32),
        pltpu.SemaphoreType.DMA,
        pltpu.SemaphoreType.DMA,
        pltpu.SemaphoreType.DMA,
        pltpu.SemaphoreType.DMA,
    ],
)
def _embed_gather(ids_hbm, table_hbm, out_hbm, idx_v, rows_v, gsem_a, gsem_b, osem_a, osem_b):
    wid = lax.axis_index("s") * 2 + lax.axis_index("c")
    base = wid * ROWS_PER_WORKER
    pltpu.sync_copy(ids_hbm.at[wid], idx_v)

    gsems = (gsem_a, gsem_b)
    osems = (osem_a, osem_b)
    gathers = [None] * NCHUNK
    outs = [None] * NCHUNK

    gathers[0] = pltpu.async_copy(table_hbm.at[idx_v.at[0]], rows_v.at[0], gsems[0])
    for j in range(NCHUNK):
        buf = j % NBUF
        gathers[j].wait()
        if j + 1 < NCHUNK:
            nbuf = (j + 1) % NBUF
            if j >= 1:
                outs[j - 1].wait()
            gathers[j + 1] = pltpu.async_copy(
                table_hbm.at[idx_v.at[j + 1]], rows_v.at[nbuf], gsems[nbuf])
        outs[j] = pltpu.async_copy(
            rows_v.at[buf], out_hbm.at[pl.ds(base + j * CHUNK, CHUNK)], osems[buf])
    outs[NCHUNK - 2].wait()
    outs[NCHUNK - 1].wait()


def kernel(input_ids, embedding_table):
    b, l = input_ids.shape
    ids = input_ids.astype(jnp.int32).reshape(NUM_WORKERS, NCHUNK, CHUNK)
    flat = _embed_gather(ids, embedding_table)
    return flat.reshape(b, l, D_MODEL)


# speedup vs baseline: 1.4986x; 1.0397x over previous
import functools

import jax
import jax.numpy as jnp
from jax import lax
from jax.experimental import pallas as pl
from jax.experimental.pallas import tpu as pltpu
from jax.experimental.pallas import tpu_sc as plsc

D_MODEL = 768
N_TOKENS = 4 * 2048
NUM_WORKERS = 32
ROWS_PER_WORKER = N_TOKENS // NUM_WORKERS
CHUNK = 32
NCHUNK = ROWS_PER_WORKER // CHUNK
NBUF = 4

_MESH = plsc.VectorSubcoreMesh(core_axis_name="c", subcore_axis_name="s")


@functools.partial(
    pl.kernel,
    mesh=_MESH,
    out_type=jax.ShapeDtypeStruct((N_TOKENS, D_MODEL), jnp.float32),
    scratch_types=[
        pltpu.VMEM((ROWS_PER_WORKER,), jnp.int32),
        pltpu.VMEM((NBUF, CHUNK, D_MODEL), jnp.float32),
    ]
    + [pltpu.SemaphoreType.DMA] * (2 * NBUF),
)
def _embed_gather(ids_hbm, table_hbm, out_hbm, idx_v, rows_v, *sems):
    gsems, osems = sems[:NBUF], sems[NBUF:]
    wid = lax.axis_index("s") * 2 + lax.axis_index("c")
    base = wid * ROWS_PER_WORKER
    pltpu.sync_copy(ids_hbm.at[pl.ds(base, ROWS_PER_WORKER)], idx_v)

    gathers = [None] * NCHUNK
    outs = [None] * NCHUNK

    def start_gather(j):
        b = j % NBUF
        gathers[j] = pltpu.async_copy(
            table_hbm.at[idx_v.at[pl.ds(j * CHUNK, CHUNK)]], rows_v.at[b], gsems[b])

    for j in range(min(NBUF - 1, NCHUNK)):
        start_gather(j)

    for j in range(NCHUNK):
        gathers[j].wait()
        outs[j] = pltpu.async_copy(
            rows_v.at[j % NBUF], out_hbm.at[pl.ds(base + j * CHUNK, CHUNK)],
            osems[j % NBUF])
        nj = j + NBUF - 1
        if nj < NCHUNK:
            if nj - NBUF >= 0:
                outs[nj - NBUF].wait()
            start_gather(nj)

    for j in range(max(0, NCHUNK - NBUF), NCHUNK):
        outs[j].wait()


def kernel(input_ids, embedding_table):
    b, l = input_ids.shape
    ids = input_ids.astype(jnp.int32).reshape(N_TOKENS)
    flat = _embed_gather(ids, embedding_table)
    return flat.reshape(b, l, D_MODEL)


# speedup vs baseline: 1.5230x; 1.0163x over previous
import functools

import jax
import jax.numpy as jnp
from jax import lax
from jax.experimental import pallas as pl
from jax.experimental.pallas import tpu as pltpu
from jax.experimental.pallas import tpu_sc as plsc

D_MODEL = 768
N_TOKENS = 4 * 2048
NUM_WORKERS = 32
ROWS_PER_WORKER = N_TOKENS // NUM_WORKERS
CHUNK = 16
NCHUNK = ROWS_PER_WORKER // CHUNK
NBUF = 8

_MESH = plsc.VectorSubcoreMesh(core_axis_name="c", subcore_axis_name="s")


@functools.partial(
    pl.kernel,
    mesh=_MESH,
    out_type=jax.ShapeDtypeStruct((N_TOKENS, D_MODEL), jnp.float32),
    scratch_types=[
        pltpu.VMEM((ROWS_PER_WORKER,), jnp.int32),
        pltpu.VMEM((NBUF, CHUNK, D_MODEL), jnp.float32),
    ]
    + [pltpu.SemaphoreType.DMA] * (2 * NBUF),
)
def _embed_gather(ids_hbm, table_hbm, out_hbm, idx_v, rows_v, *sems):
    gsems, osems = sems[:NBUF], sems[NBUF:]
    wid = lax.axis_index("s") * 2 + lax.axis_index("c")
    base = wid * ROWS_PER_WORKER
    pltpu.sync_copy(ids_hbm.at[pl.ds(base, ROWS_PER_WORKER)], idx_v)

    gathers = [None] * NCHUNK
    outs = [None] * NCHUNK

    def start_gather(j):
        b = j % NBUF
        gathers[j] = pltpu.async_copy(
            table_hbm.at[idx_v.at[pl.ds(j * CHUNK, CHUNK)]], rows_v.at[b], gsems[b])

    for j in range(min(NBUF - 1, NCHUNK)):
        start_gather(j)

    for j in range(NCHUNK):
        gathers[j].wait()
        outs[j] = pltpu.async_copy(
            rows_v.at[j % NBUF], out_hbm.at[pl.ds(base + j * CHUNK, CHUNK)],
            osems[j % NBUF])
        nj = j + NBUF - 1
        if nj < NCHUNK:
            if nj - NBUF >= 0:
                outs[nj - NBUF].wait()
            start_gather(nj)

    for j in range(max(0, NCHUNK - NBUF), NCHUNK):
        outs[j].wait()


def kernel(input_ids, embedding_table):
    b, l = input_ids.shape
    ids = input_ids.astype(jnp.int32).reshape(N_TOKENS)
    flat = _embed_gather(ids, embedding_table)
    return flat.reshape(b, l, D_MODEL)


# speedup vs baseline: 1.5321x; 1.0059x over previous
import functools

import jax
import jax.numpy as jnp
from jax import lax
from jax.experimental import pallas as pl
from jax.experimental.pallas import tpu as pltpu
from jax.experimental.pallas import tpu_sc as plsc

D_MODEL = 768
N_TOKENS = 4 * 2048
NUM_WORKERS = 32
ROWS_PER_WORKER = N_TOKENS // NUM_WORKERS
CHUNK = 16
NCHUNK = ROWS_PER_WORKER // CHUNK
NBUF = 8
OUT_MERGE = 2
NPAIR = NBUF // OUT_MERGE
RING_ROWS = NBUF * CHUNK

_MESH = plsc.VectorSubcoreMesh(core_axis_name="c", subcore_axis_name="s")


@functools.partial(
    pl.kernel,
    mesh=_MESH,
    out_type=jax.ShapeDtypeStruct((N_TOKENS, D_MODEL), jnp.float32),
    scratch_types=[
        pltpu.VMEM((ROWS_PER_WORKER,), jnp.int32),
        pltpu.VMEM((RING_ROWS, D_MODEL), jnp.float32),
    ]
    + [pltpu.SemaphoreType.DMA] * (NBUF + NPAIR),
)
def _embed_gather(ids_hbm, table_hbm, out_hbm, idx_v, rows_v, *sems):
    gsems, osems = sems[:NBUF], sems[NBUF:]
    wid = lax.axis_index("s") * 2 + lax.axis_index("c")
    base = wid * ROWS_PER_WORKER
    pltpu.sync_copy(ids_hbm.at[pl.ds(base, ROWS_PER_WORKER)], idx_v)

    gathers = [None] * NCHUNK
    outs = {}
    out_waited = set()

    def start_gather(j):
        b = j % NBUF
        gathers[j] = pltpu.async_copy(
            table_hbm.at[idx_v.at[pl.ds(j * CHUNK, CHUNK)]],
            rows_v.at[pl.ds(b * CHUNK, CHUNK)], gsems[b])

    def ensure_out_done(p):
        if p in outs and p not in out_waited:
            outs[p].wait()
            out_waited.add(p)

    for j in range(min(NBUF - 1, NCHUNK)):
        start_gather(j)

    for j in range(NCHUNK):
        gathers[j].wait()
        if j % OUT_MERGE == OUT_MERGE - 1:
            p = j // OUT_MERGE
            b0 = (j - OUT_MERGE + 1) % NBUF
            outs[p] = pltpu.async_copy(
                rows_v.at[pl.ds(b0 * CHUNK, OUT_MERGE * CHUNK)],
                out_hbm.at[pl.ds(base + (j - OUT_MERGE + 1) * CHUNK,
                                 OUT_MERGE * CHUNK)],
                osems[p % NPAIR])
        nj = j + NBUF - 1
        if nj < NCHUNK:
            prev = nj - NBUF
            if prev >= 0:
                ensure_out_done(prev // OUT_MERGE)
            start_gather(nj)

    for p in range(NCHUNK // OUT_MERGE):
        ensure_out_done(p)


def kernel(input_ids, embedding_table):
    b, l = input_ids.shape
    ids = input_ids.astype(jnp.int32).reshape(N_TOKENS)
    flat = _embed_gather(ids, embedding_table)
    return flat.reshape(b, l, D_MODEL)
